# baseline (speedup 1.0000x reference)
import functools

import jax
import jax.numpy as jnp
from jax import lax
from jax.experimental import pallas as pl
from jax.experimental.pallas import tpu as pltpu
from jax.experimental.pallas import tpu_sc as plsc

BATCH = 16384
V = 1000000

try:
    _INFO = plsc.get_sparse_core_info()
    _NC = _INFO.num_cores
    _NS = _INFO.num_subcores
    _L = _INFO.num_lanes
except Exception:
    _NC, _NS, _L = 2, 16, 16
_NW = _NC * _NS
_BPW = BATCH // _NW
_NSL = _BPW // _L

_SPAN = 124992
_CH = 2048
_NCH = 61
_TAIL = 64
_TAIL7 = 128

_mesh = plsc.VectorSubcoreMesh(core_axis_name="c", subcore_axis_name="s")


@functools.partial(
    pl.kernel,
    out_type=(
        jax.ShapeDtypeStruct((_NW, 8, 128), jnp.int32),
        jax.ShapeDtypeStruct((_NW, 8, 128), jnp.int32),
        jax.ShapeDtypeStruct((V,), jnp.float32),
        jax.ShapeDtypeStruct((V,), jnp.float32),
        jax.ShapeDtypeStruct((V,), jnp.float32),
        jax.ShapeDtypeStruct((V,), jnp.float32),
    ),
    mesh=_mesh,
    compiler_params=pltpu.CompilerParams(
        needs_layout_passes=False, use_tc_tiling_on_sc=False
    ),
    scratch_types=[
        pltpu.VMEM((_BPW, 2), jnp.int32),
        pltpu.VMEM((4, 128), jnp.int32),
        pltpu.VMEM((4, 128), jnp.int32),
        pltpu.VMEM((_CH, 1), jnp.float32),
        pltpu.VMEM((_CH,), jnp.float32),
        pltpu.SemaphoreType.DMA,
    ],
)
def _fm_stage1(inp_hbm, ub_hbm, ib_hbm, ue_hbm, ie_hbm,
               uix_hbm, iix_hbm, ub1_hbm, ib1_hbm, ue1_hbm, ie1_hbm,
               inp_v, uc_v, ic_v, c2_v, c1_v, sem):
    wid = lax.axis_index("s") * _NC + lax.axis_index("c")
    base = wid * _BPW

    pltpu.sync_copy(inp_hbm.at[pl.ds(base, _BPW)], inp_v)
    iota = lax.iota(jnp.int32, _L)
    zeros = iota * 0
    ones = zeros + 1
    for s in range(_NSL):
        rows = iota + _L * s
        u = plsc.load_gather(inp_v, [rows, zeros])
        it = plsc.load_gather(inp_v, [rows, ones])
        uc_v[s // 8, pl.ds((s % 8) * _L, _L)] = u
        ic_v[s // 8, pl.ds((s % 8) * _L, _L)] = it
    for j in range(4):
        pltpu.sync_copy(uc_v.at[j], uix_hbm.at[wid, j])
        pltpu.sync_copy(ic_v.at[j], iix_hbm.at[wid, j])

    tid = wid // 8
    j = wid % 8
    start = j * _SPAN

    def squeeze_chunk(n):
        for s in range(n // _L):
            rows = iota + _L * s
            c1_v[pl.ds(_L * s, _L)] = plsc.load_gather(c2_v, [rows, zeros])

    def do_span(tab, out1):
        def body(c, carry):
            off = start + _CH * c
            pltpu.sync_copy(tab.at[pl.ds(off, _CH)], c2_v)
            squeeze_chunk(_CH)
            pltpu.sync_copy(c1_v, out1.at[pl.ds(off, _CH)])
            return carry

        lax.fori_loop(0, _NCH, body, 0)
        toff = start + _NCH * _CH

        @pl.when(j < 7)
        def _():
            pltpu.sync_copy(tab.at[pl.ds(toff, _TAIL)], c2_v.at[pl.ds(0, _TAIL)])
            squeeze_chunk(_TAIL)
            pltpu.sync_copy(c1_v.at[pl.ds(0, _TAIL)], out1.at[pl.ds(toff, _TAIL)])

        @pl.when(j == 7)
        def _():
            pltpu.sync_copy(tab.at[pl.ds(toff, _TAIL7)], c2_v.at[pl.ds(0, _TAIL7)])
            squeeze_chunk(_TAIL7)
            pltpu.sync_copy(c1_v.at[pl.ds(0, _TAIL7)], out1.at[pl.ds(toff, _TAIL7)])

    for t, (tab, out1) in enumerate(
        [(ub_hbm, ub1_hbm), (ib_hbm, ib1_hbm), (ue_hbm, ue1_hbm), (ie_hbm, ie1_hbm)]
    ):
        @pl.when(tid == t)
        def _(tab=tab, out1=out1):
            do_span(tab, out1)


@functools.partial(
    pl.kernel,
    out_type=jax.ShapeDtypeStruct((BATCH,), jnp.float32),
    mesh=_mesh,
    compiler_params=pltpu.CompilerParams(needs_layout_passes=False),
    scratch_types=[
        pltpu.VMEM((8, 128), jnp.int32),
        pltpu.VMEM((8, 128), jnp.int32),
        pltpu.VMEM((_BPW,), jnp.float32),
        pltpu.VMEM((_BPW,), jnp.float32),
        pltpu.VMEM((_BPW,), jnp.float32),
        pltpu.VMEM((_BPW,), jnp.float32),
        pltpu.VMEM((_L,), jnp.float32),
        pltpu.VMEM((_BPW,), jnp.float32),
        pltpu.SemaphoreType.DMA,
    ],
)
def _fm_stage2(uix_hbm, iix_hbm, ub1_hbm, ib1_hbm, ue1_hbm, ie1_hbm, w0_hbm,
               out_hbm, uc_v, ic_v, ub_v, ib_v, ue_v, ie_v, w0_v, out_v, sem):
    wid = lax.axis_index("s") * _NC + lax.axis_index("c")
    base = wid * _BPW

    pltpu.sync_copy(uix_hbm.at[wid], uc_v)
    pltpu.sync_copy(iix_hbm.at[wid], ic_v)
    pltpu.sync_copy(w0_hbm, w0_v)

    copies = []
    for j in range(4):
        sl = pl.ds(j * 128, 128)
        copies.append(pltpu.async_copy(ub1_hbm.at[uc_v.at[j]], ub_v.at[sl], sem))
        copies.append(pltpu.async_copy(ib1_hbm.at[ic_v.at[j]], ib_v.at[sl], sem))
        copies.append(pltpu.async_copy(ue1_hbm.at[uc_v.at[j]], ue_v.at[sl], sem))
        copies.append(pltpu.async_copy(ie1_hbm.at[ic_v.at[j]], ie_v.at[sl], sem))
    for c in copies:
        c.wait()

    w0r = w0_v[...]
    for s in range(_NSL):
        sl = pl.ds(s * _L, _L)
        out_v[sl] = w0r + ub_v[sl] + ib_v[sl] + ue_v[sl] * ie_v[sl]

    pltpu.sync_copy(out_v, out_hbm.at[pl.ds(base, _BPW)])


def kernel(INPUT, userBias, itemBias, userEmbed, itemEmbed, w0):
    uix, iix, ub1, ib1, ue1, ie1 = _fm_stage1(
        INPUT.astype(jnp.int32), userBias, itemBias, userEmbed, itemEmbed
    )
    out = _fm_stage2(
        uix, iix, ub1, ib1, ue1, ie1,
        jnp.broadcast_to(w0.reshape(()), (_L,)),
    )
    return out.reshape(BATCH, 1)


# speedup vs baseline: 58.5998x; 58.5998x over previous
import functools

import jax
import jax.numpy as jnp
from jax import lax
from jax.experimental import pallas as pl
from jax.experimental.pallas import tpu as pltpu
from jax.experimental.pallas import tpu_sc as plsc

BATCH = 16384
_VPAD = 1000448

try:
    _INFO = plsc.get_sparse_core_info()
    _NC = _INFO.num_cores
    _NS = _INFO.num_subcores
    _L = _INFO.num_lanes
except Exception:
    _NC, _NS, _L = 2, 16, 16
_NW = _NC * _NS
_BPW = BATCH // _NW
_CHUNK = 128
_NCHUNK = _BPW // _CHUNK
_NSL = _BPW // _L

_mesh = plsc.VectorSubcoreMesh(core_axis_name="c", subcore_axis_name="s")


@functools.partial(
    pl.kernel,
    out_type=jax.ShapeDtypeStruct((BATCH,), jnp.float32),
    mesh=_mesh,
    compiler_params=pltpu.CompilerParams(needs_layout_passes=False),
    scratch_types=[
        pltpu.VMEM((2 * _BPW,), jnp.int32),
        pltpu.VMEM((_NCHUNK, _CHUNK), jnp.int32),
        pltpu.VMEM((_NCHUNK, _CHUNK), jnp.int32),
        pltpu.VMEM((_BPW,), jnp.float32),
        pltpu.VMEM((_BPW,), jnp.float32),
        pltpu.VMEM((_BPW,), jnp.float32),
        pltpu.VMEM((_BPW,), jnp.float32),
        pltpu.VMEM((_L,), jnp.float32),
        pltpu.VMEM((_BPW,), jnp.float32),
        pltpu.SemaphoreType.DMA,
    ],
)
def _fm_sc(inp_hbm, ub_hbm, ib_hbm, ue_hbm, ie_hbm, w0_hbm, out_hbm,
           inp_v, uidx_v, iidx_v, ub_v, ib_v, ue_v, ie_v, w0_v, out_v, sem):
    wid = lax.axis_index("s") * _NC + lax.axis_index("c")
    base = wid * _BPW

    pltpu.sync_copy(inp_hbm.at[pl.ds(2 * base, 2 * _BPW)], inp_v)
    pltpu.sync_copy(w0_hbm, w0_v)

    iota2 = lax.iota(jnp.int32, _L) * 2
    for j in range(_NSL):
        upos = iota2 + (2 * _L) * j
        u = plsc.load_gather(inp_v, [upos])
        it = plsc.load_gather(inp_v, [upos + 1])
        uidx_v[j // 8, pl.ds((j % 8) * _L, _L)] = u
        iidx_v[j // 8, pl.ds((j % 8) * _L, _L)] = it

    copies = []
    for j in range(_NCHUNK):
        sl = pl.ds(j * _CHUNK, _CHUNK)
        copies.append(pltpu.async_copy(ub_hbm.at[uidx_v.at[j]], ub_v.at[sl], sem))
        copies.append(pltpu.async_copy(ib_hbm.at[iidx_v.at[j]], ib_v.at[sl], sem))
        copies.append(pltpu.async_copy(ue_hbm.at[uidx_v.at[j]], ue_v.at[sl], sem))
        copies.append(pltpu.async_copy(ie_hbm.at[iidx_v.at[j]], ie_v.at[sl], sem))
    for c in copies:
        c.wait()

    w0r = w0_v[...]
    for j in range(_NSL):
        sl = pl.ds(j * _L, _L)
        out_v[sl] = w0r + ub_v[sl] + ib_v[sl] + ue_v[sl] * ie_v[sl]

    pltpu.sync_copy(out_v, out_hbm.at[pl.ds(base, _BPW)])


def _flat(table):
    n = table.shape[0]
    return jnp.pad(table, ((0, _VPAD - n), (0, 0))).reshape(-1)


def kernel(INPUT, userBias, itemBias, userEmbed, itemEmbed, w0):
    out = _fm_sc(
        INPUT.astype(jnp.int32).reshape(-1),
        _flat(userBias),
        _flat(itemBias),
        _flat(userEmbed),
        _flat(itemEmbed),
        jnp.broadcast_to(w0.reshape(()), (_L,)),
    )
    return out.reshape(BATCH, 1)


# speedup vs baseline: 67.6738x; 1.1548x over previous
import functools

import jax
import jax.numpy as jnp
from jax import lax
from jax.experimental import pallas as pl
from jax.experimental.pallas import tpu as pltpu
from jax.experimental.pallas import tpu_sc as plsc

BATCH = 16384
_VPAD = 1000448

try:
    _INFO = plsc.get_sparse_core_info()
    _NC = _INFO.num_cores
    _NS = _INFO.num_subcores
    _L = _INFO.num_lanes
except Exception:
    _NC, _NS, _L = 2, 16, 16
_NW = _NC * _NS
_BPW = BATCH // _NW
_CHUNK = 128
_NCHUNK = _BPW // _CHUNK
_NSL = _BPW // _L

_mesh = plsc.VectorSubcoreMesh(core_axis_name="c", subcore_axis_name="s")


@functools.partial(
    pl.kernel,
    out_type=jax.ShapeDtypeStruct((BATCH,), jnp.float32),
    mesh=_mesh,
    compiler_params=pltpu.CompilerParams(needs_layout_passes=False),
    scratch_types=[
        pltpu.VMEM((_BPW,), jnp.int32),
        pltpu.VMEM((_BPW,), jnp.int32),
        pltpu.VMEM((_NCHUNK, _CHUNK), jnp.int32),
        pltpu.VMEM((_NCHUNK, _CHUNK), jnp.int32),
        pltpu.VMEM((_BPW,), jnp.float32),
        pltpu.VMEM((_BPW,), jnp.float32),
        pltpu.VMEM((_BPW,), jnp.float32),
        pltpu.VMEM((_BPW,), jnp.float32),
        pltpu.VMEM((_L,), jnp.float32),
        pltpu.VMEM((_BPW,), jnp.float32),
        pltpu.SemaphoreType.DMA,
    ],
)
def _fm_sc(u_hbm, i_hbm, ub_hbm, ib_hbm, ue_hbm, ie_hbm, w0_hbm, out_hbm,
           u_v, i_v, uidx_v, iidx_v, ub_v, ib_v, ue_v, ie_v, w0_v, out_v, sem):
    wid = lax.axis_index("s") * _NC + lax.axis_index("c")
    base = wid * _BPW

    pltpu.sync_copy(u_hbm.at[pl.ds(base, _BPW)], u_v)
    pltpu.sync_copy(i_hbm.at[pl.ds(base, _BPW)], i_v)
    pltpu.sync_copy(w0_hbm, w0_v)

    for j in range(_NSL):
        sl = pl.ds(j * _L, _L)
        uidx_v[j // 8, pl.ds((j % 8) * _L, _L)] = u_v[sl]
        iidx_v[j // 8, pl.ds((j % 8) * _L, _L)] = i_v[sl]

    copies = []
    for j in range(_NCHUNK):
        sl = pl.ds(j * _CHUNK, _CHUNK)
        copies.append(pltpu.async_copy(ub_hbm.at[uidx_v.at[j]], ub_v.at[sl], sem))
        copies.append(pltpu.async_copy(ib_hbm.at[iidx_v.at[j]], ib_v.at[sl], sem))
        copies.append(pltpu.async_copy(ue_hbm.at[uidx_v.at[j]], ue_v.at[sl], sem))
        copies.append(pltpu.async_copy(ie_hbm.at[iidx_v.at[j]], ie_v.at[sl], sem))
    for c in copies:
        c.wait()

    w0r = w0_v[...]
    for j in range(_NSL):
        sl = pl.ds(j * _L, _L)
        out_v[sl] = w0r + ub_v[sl] + ib_v[sl] + ue_v[sl] * ie_v[sl]

    pltpu.sync_copy(out_v, out_hbm.at[pl.ds(base, _BPW)])


def _flat(table):
    n = table.shape[0]
    return jnp.pad(table, ((0, _VPAD - n), (0, 0))).reshape(-1)


def kernel(INPUT, userBias, itemBias, userEmbed, itemEmbed, w0):
    ids = INPUT.astype(jnp.int32)
    out = _fm_sc(
        ids[:, 0],
        ids[:, 1],
        _flat(userBias),
        _flat(itemBias),
        _flat(userEmbed),
        _flat(itemEmbed),
        jnp.broadcast_to(w0.reshape(()), (_L,)),
    )
    return out.reshape(BATCH, 1)
